# baseline (speedup 1.0000x reference)
import functools

import jax
import jax.numpy as jnp
from jax import lax
from jax.experimental import pallas as pl
from jax.experimental.pallas import tpu as pltpu
from jax.experimental.pallas import tpu_sc as plsc

B = 16384
OBS = 80
F = 64
V = 8
HID = 10
NUM_OUT = 30

NC = 2
NS = 16
L = 16
NW = NC * NS
ROWS_PER_W = B // NW
CHUNKS = ROWS_PER_W // L
NPAIR = F // 2
PT_ROWS = NPAIR * V * V
UNROLL = 2


def _body(xt_h, pt_h, misc_h, logits_h, value_h,
          xt_v, pt_v, misc_v, logits_v, value_v):
    wid = lax.axis_index("s") * NC + lax.axis_index("c")
    base = wid * ROWS_PER_W

    pltpu.sync_copy(xt_h.at[:, pl.ds(base, ROWS_PER_W)], xt_v)
    pltpu.sync_copy(pt_h, pt_v)
    pltpu.sync_copy(misc_h, misc_v)

    ptc = [pt_v.at[pl.ds(c * PT_ROWS, PT_ROWS)] for c in range(HID)]

    btr = misc_v[pl.ds(0 * 32, L)]
    w64 = misc_v[pl.ds(1 * 32, L)]
    w65 = misc_v[pl.ds(2 * 32, L)]
    whA = [misc_v[pl.ds((3 + c) * 32, L)] for c in range(HID)]
    whB = [misc_v[pl.ds((3 + c) * 32 + L, L)] for c in range(HID)]
    bhA = misc_v[pl.ds(13 * 32, L)]
    bhB = misc_v[pl.ds(13 * 32 + L, L)]
    wv = misc_v[pl.ds(14 * 32, L)]

    def trunk_for(i):
        def xcol(c):
            return xt_v[c, pl.ds(i * L, L)]

        x64 = xcol(F)
        x65 = xcol(F + 1)
        acc = [btr[c] + x64 * w64[c] + x65 * w65[c] for c in range(HID)]
        for p in range(NPAIR):
            xe = xcol(2 * p).astype(jnp.int32)
            xo = xcol(2 * p + 1).astype(jnp.int32)
            ridx = (p * V * V) + xe * V + xo
            g = [plsc.load_gather(ptc[c], [ridx]) for c in range(HID)]
            acc = [a + gc for a, gc in zip(acc, g)]
        return [jnp.maximum(a, 0.0) for a in acc]

    def pair_step(j, carry):
        i0 = j * UNROLL
        trunks = [trunk_for(i0 + u) for u in range(UNROLL)]
        for o in range(NUM_OUT):
            if o < L:
                bo = jnp.broadcast_to(bhA[o], (L,))
                wo = [jnp.broadcast_to(whA[c][o], (L,)) for c in range(HID)]
            else:
                bo = jnp.broadcast_to(bhB[o - 14], (L,))
                wo = [jnp.broadcast_to(whB[c][o - 14], (L,)) for c in range(HID)]
            for u in range(UNROLL):
                lo = bo
                for c in range(HID):
                    lo = lo + trunks[u][c] * wo[c]
                logits_v[o, pl.ds((i0 + u) * L, L)] = lo
        wvb = [jnp.broadcast_to(wv[c], (L,)) for c in range(HID)]
        bvb = jnp.broadcast_to(wv[HID], (L,))
        for u in range(UNROLL):
            z = bvb
            for c in range(HID):
                z = z + trunks[u][c] * wvb[c]
            e = jnp.exp(2.0 * z)
            value_v[pl.ds((i0 + u) * L, L)] = (e - 1.0) / (e + 1.0)
        return carry

    lax.fori_loop(0, CHUNKS // UNROLL, pair_step, 0)

    pltpu.sync_copy(logits_v, logits_h.at[:, pl.ds(base, ROWS_PER_W)])
    pltpu.sync_copy(value_v, value_h.at[pl.ds(base, ROWS_PER_W)])


@functools.partial(
    pl.kernel,
    out_type=(jax.ShapeDtypeStruct((NUM_OUT, B), jnp.float32),
              jax.ShapeDtypeStruct((B,), jnp.float32)),
    mesh=plsc.VectorSubcoreMesh(core_axis_name="c", subcore_axis_name="s",
                                num_cores=NC, num_subcores=NS),
    compiler_params=pltpu.CompilerParams(needs_layout_passes=False,
                                         use_tc_tiling_on_sc=True),
    scratch_types=[
        pltpu.VMEM((OBS, ROWS_PER_W), jnp.float32),
        pltpu.VMEM((HID * PT_ROWS,), jnp.float32),
        pltpu.VMEM((16 * 32,), jnp.float32),
        pltpu.VMEM((NUM_OUT, ROWS_PER_W), jnp.float32),
        pltpu.VMEM((ROWS_PER_W,), jnp.float32),
    ],
)
def _pvnet_sc(xt_h, pt_h, misc_h, logits_h, value_h,
              xt_v, pt_v, misc_v, logits_v, value_v):
    _body(xt_h, pt_h, misc_h, logits_h, value_h,
          xt_v, pt_v, misc_v, logits_v, value_v)


def kernel(x, one_hot_indices, identity_indices, values,
           W_trunk, b_trunk, W_logits, b_logits, W_value, b_value):
    w3 = W_trunk[:F * V].reshape(F, V, HID)
    pt = (w3[0::2, :, None, :] + w3[1::2, None, :, :]).reshape(PT_ROWS, HID)
    ptcm = pt.T.reshape(HID * PT_ROWS)
    misc = jnp.zeros((16, 32), jnp.float32)
    misc = misc.at[0, :HID].set(b_trunk)
    misc = misc.at[1, :HID].set(W_trunk[F * V])
    misc = misc.at[2, :HID].set(W_trunk[F * V + 1])
    misc = misc.at[3:3 + HID, :L].set(W_logits[:, 0:L])
    misc = misc.at[3:3 + HID, L:L + L].set(W_logits[:, 14:NUM_OUT])
    misc = misc.at[13, :L].set(b_logits[0:L])
    misc = misc.at[13, L:L + L].set(b_logits[14:NUM_OUT])
    misc = misc.at[14, :HID].set(W_value[:, 0])
    misc = misc.at[14, HID].set(b_value[0])
    logits_t, value = _pvnet_sc(x.T, ptcm, misc.reshape(16 * 32))
    return logits_t.T, value.reshape(B, 1)


# speedup vs baseline: 1.3199x; 1.3199x over previous
import functools

import jax
import jax.numpy as jnp
from jax import lax
from jax.experimental import pallas as pl
from jax.experimental.pallas import tpu as pltpu
from jax.experimental.pallas import tpu_sc as plsc

B = 16384
OBS = 80
F = 64
V = 8
HID = 10
NUM_OUT = 30

NC = 2
NS = 16
L = 16
NW = NC * NS
ROWS_PER_W = B // NW
CHUNKS = ROWS_PER_W // L
NPAIR = F // 2
PT_ROWS = NPAIR * V * V
UNROLL = 2


def _body(xt_h, pt_h, misc_h, logits_h, value_h,
          xt_v, pt_v, misc_v, logits_v, value_v, tb_v):
    wid = lax.axis_index("s") * NC + lax.axis_index("c")
    base = wid * ROWS_PER_W

    pltpu.sync_copy(xt_h.at[:, pl.ds(base, ROWS_PER_W)], xt_v)
    pltpu.sync_copy(pt_h, pt_v)
    pltpu.sync_copy(misc_h, misc_v)

    ptc = [pt_v.at[pl.ds(c * PT_ROWS, PT_ROWS)] for c in range(HID)]

    btr = misc_v[pl.ds(0 * 32, L)]
    w64 = misc_v[pl.ds(1 * 32, L)]
    w65 = misc_v[pl.ds(2 * 32, L)]
    whA = [misc_v[pl.ds((3 + c) * 32, L)] for c in range(HID)]
    whB = [misc_v[pl.ds((3 + c) * 32 + L, L)] for c in range(HID)]
    bhA = misc_v[pl.ds(13 * 32, L)]
    bhB = misc_v[pl.ds(13 * 32 + L, L)]
    wv = misc_v[pl.ds(14 * 32, L)]

    def trunk_for(i):
        def xcol(c):
            return xt_v[c, pl.ds(i * L, L)]

        x64 = xcol(F)
        x65 = xcol(F + 1)
        acc = [btr[c] + x64 * w64[c] + x65 * w65[c] for c in range(HID)]
        for p in range(NPAIR):
            xe = xcol(2 * p).astype(jnp.int32)
            xo = xcol(2 * p + 1).astype(jnp.int32)
            ridx = (p * V * V) + xe * V + xo
            g = [plsc.load_gather(ptc[c], [ridx]) for c in range(HID)]
            acc = [a + gc for a, gc in zip(acc, g)]
        return [jnp.maximum(a, 0.0) for a in acc]

    def pass1(i, carry):
        trunk = trunk_for(i)
        for c in range(HID):
            tb_v[pl.ds(c * ROWS_PER_W + i * L, L)] = trunk[c]
        z = jnp.broadcast_to(wv[HID], (L,))
        for c in range(HID):
            z = z + trunk[c] * jnp.broadcast_to(wv[c], (L,))
        e = jnp.exp(2.0 * z)
        value_v[pl.ds(i * L, L)] = (e - 1.0) / (e + 1.0)
        return carry

    lax.fori_loop(0, CHUNKS, pass1, 0)

    def pass2(i, carry):
        trunk = [tb_v[pl.ds(c * ROWS_PER_W + i * L, L)] for c in range(HID)]
        for o in range(NUM_OUT):
            if o < L:
                lo = jnp.broadcast_to(bhA[o], (L,))
                for c in range(HID):
                    lo = lo + trunk[c] * jnp.broadcast_to(whA[c][o], (L,))
            else:
                lo = jnp.broadcast_to(bhB[o - 14], (L,))
                for c in range(HID):
                    lo = lo + trunk[c] * jnp.broadcast_to(whB[c][o - 14], (L,))
            logits_v[o, pl.ds(i * L, L)] = lo
        return carry

    lax.fori_loop(0, CHUNKS, pass2, 0)

    pltpu.sync_copy(logits_v, logits_h.at[:, pl.ds(base, ROWS_PER_W)])
    pltpu.sync_copy(value_v, value_h.at[pl.ds(base, ROWS_PER_W)])


@functools.partial(
    pl.kernel,
    out_type=(jax.ShapeDtypeStruct((NUM_OUT, B), jnp.float32),
              jax.ShapeDtypeStruct((B,), jnp.float32)),
    mesh=plsc.VectorSubcoreMesh(core_axis_name="c", subcore_axis_name="s",
                                num_cores=NC, num_subcores=NS),
    compiler_params=pltpu.CompilerParams(needs_layout_passes=False,
                                         use_tc_tiling_on_sc=True),
    scratch_types=[
        pltpu.VMEM((OBS, ROWS_PER_W), jnp.float32),
        pltpu.VMEM((HID * PT_ROWS,), jnp.float32),
        pltpu.VMEM((16 * 32,), jnp.float32),
        pltpu.VMEM((NUM_OUT, ROWS_PER_W), jnp.float32),
        pltpu.VMEM((ROWS_PER_W,), jnp.float32),
        pltpu.VMEM((HID * ROWS_PER_W,), jnp.float32),
    ],
)
def _pvnet_sc(xt_h, pt_h, misc_h, logits_h, value_h,
              xt_v, pt_v, misc_v, logits_v, value_v, tb_v):
    _body(xt_h, pt_h, misc_h, logits_h, value_h,
          xt_v, pt_v, misc_v, logits_v, value_v, tb_v)


def kernel(x, one_hot_indices, identity_indices, values,
           W_trunk, b_trunk, W_logits, b_logits, W_value, b_value):
    w3 = W_trunk[:F * V].reshape(F, V, HID)
    pt = (w3[0::2, :, None, :] + w3[1::2, None, :, :]).reshape(PT_ROWS, HID)
    ptcm = pt.T.reshape(HID * PT_ROWS)
    misc = jnp.zeros((16, 32), jnp.float32)
    misc = misc.at[0, :HID].set(b_trunk)
    misc = misc.at[1, :HID].set(W_trunk[F * V])
    misc = misc.at[2, :HID].set(W_trunk[F * V + 1])
    misc = misc.at[3:3 + HID, :L].set(W_logits[:, 0:L])
    misc = misc.at[3:3 + HID, L:L + L].set(W_logits[:, 14:NUM_OUT])
    misc = misc.at[13, :L].set(b_logits[0:L])
    misc = misc.at[13, L:L + L].set(b_logits[14:NUM_OUT])
    misc = misc.at[14, :HID].set(W_value[:, 0])
    misc = misc.at[14, HID].set(b_value[0])
    logits_t, value = _pvnet_sc(x.T, ptcm, misc.reshape(16 * 32))
    return logits_t.T, value.reshape(B, 1)


# speedup vs baseline: 1.4384x; 1.0898x over previous
import functools

import jax
import jax.numpy as jnp
from jax import lax
from jax.experimental import pallas as pl
from jax.experimental.pallas import tpu as pltpu
from jax.experimental.pallas import tpu_sc as plsc

B = 16384
OBS = 80
F = 64
V = 8
HID = 10
NUM_OUT = 30

NC = 2
NS = 16
L = 16
NW = NC * NS
ROWS_PER_W = B // NW
CHUNKS = ROWS_PER_W // L
NPAIR = F // 2
PT_ROWS = NPAIR * V * V
UNROLL = 2


def _body(xt_h, pt_h, misc_h, logits_h, value_h,
          xt_v, pt_v, misc_v, logits_v, value_v):
    wid = lax.axis_index("s") * NC + lax.axis_index("c")
    base = wid * ROWS_PER_W

    pltpu.sync_copy(xt_h.at[:, pl.ds(base, ROWS_PER_W)], xt_v)
    pltpu.sync_copy(pt_h, pt_v)
    pltpu.sync_copy(misc_h, misc_v)

    ptc = [pt_v.at[pl.ds(c * PT_ROWS, PT_ROWS)] for c in range(HID)]

    btr = misc_v[pl.ds(0 * 32, L)]
    w64 = misc_v[pl.ds(1 * 32, L)]
    w65 = misc_v[pl.ds(2 * 32, L)]
    whA = [misc_v[pl.ds((3 + c) * 32, L)] for c in range(HID)]
    whB = [misc_v[pl.ds((3 + c) * 32 + L, L)] for c in range(HID)]
    bhA = misc_v[pl.ds(13 * 32, L)]
    bhB = misc_v[pl.ds(13 * 32 + L, L)]
    wv = misc_v[pl.ds(14 * 32, L)]

    def trunk_for(i):
        def xcol(c):
            return xt_v[c, pl.ds(i * L, L)]

        x64 = xcol(F)
        x65 = xcol(F + 1)
        acc = [btr[c] + x64 * w64[c] + x65 * w65[c] for c in range(HID)]
        for p in range(NPAIR):
            xe = xcol(2 * p).astype(jnp.int32)
            xo = xcol(2 * p + 1).astype(jnp.int32)
            ridx = (p * V * V) + xe * V + xo
            g = [plsc.load_gather(ptc[c], [ridx]) for c in range(HID)]
            acc = [a + gc for a, gc in zip(acc, g)]
        return [jnp.maximum(a, 0.0) for a in acc]

    def pair_step(j, carry):
        i0 = j * UNROLL
        trunks = [trunk_for(i0 + u) for u in range(UNROLL)]
        for o in range(NUM_OUT):
            if o < L:
                bo = jnp.broadcast_to(bhA[o], (L,))
                wo = [jnp.broadcast_to(whA[c][o], (L,)) for c in range(HID)]
            else:
                bo = jnp.broadcast_to(bhB[o - 14], (L,))
                wo = [jnp.broadcast_to(whB[c][o - 14], (L,)) for c in range(HID)]
            for u in range(UNROLL):
                lo = bo
                for c in range(HID):
                    lo = lo + trunks[u][c] * wo[c]
                logits_v[o, pl.ds((i0 + u) * L, L)] = lo
        wvb = [jnp.broadcast_to(wv[c], (L,)) for c in range(HID)]
        bvb = jnp.broadcast_to(wv[HID], (L,))
        for u in range(UNROLL):
            z = bvb
            for c in range(HID):
                z = z + trunks[u][c] * wvb[c]
            e = jnp.exp(2.0 * z)
            value_v[pl.ds((i0 + u) * L, L)] = (e - 1.0) / (e + 1.0)
        return carry

    lax.fori_loop(0, CHUNKS // UNROLL, pair_step, 0)

    pltpu.sync_copy(logits_v, logits_h.at[:, pl.ds(base, ROWS_PER_W)])
    pltpu.sync_copy(value_v, value_h.at[pl.ds(base, ROWS_PER_W)])


@functools.partial(
    pl.kernel,
    out_type=(jax.ShapeDtypeStruct((NUM_OUT, B), jnp.float32),
              jax.ShapeDtypeStruct((B,), jnp.float32)),
    mesh=plsc.VectorSubcoreMesh(core_axis_name="c", subcore_axis_name="s",
                                num_cores=NC, num_subcores=NS),
    compiler_params=pltpu.CompilerParams(needs_layout_passes=False,
                                         use_tc_tiling_on_sc=True),
    scratch_types=[
        pltpu.VMEM((OBS, ROWS_PER_W), jnp.float32),
        pltpu.VMEM((HID * PT_ROWS,), jnp.float32),
        pltpu.VMEM((16 * 32,), jnp.float32),
        pltpu.VMEM((NUM_OUT, ROWS_PER_W), jnp.float32),
        pltpu.VMEM((ROWS_PER_W,), jnp.float32),
    ],
)
def _pvnet_sc(xt_h, pt_h, misc_h, logits_h, value_h,
              xt_v, pt_v, misc_v, logits_v, value_v):
    _body(xt_h, pt_h, misc_h, logits_h, value_h,
          xt_v, pt_v, misc_v, logits_v, value_v)


def kernel(x, one_hot_indices, identity_indices, values,
           W_trunk, b_trunk, W_logits, b_logits, W_value, b_value):
    w3 = W_trunk[:F * V].reshape(F, V, HID)
    pt = (w3[0::2, :, None, :] + w3[1::2, None, :, :]).reshape(PT_ROWS, HID)
    ptcm = pt.T.reshape(HID * PT_ROWS)
    misc = jnp.zeros((16, 32), jnp.float32)
    misc = misc.at[0, :HID].set(b_trunk)
    misc = misc.at[1, :HID].set(W_trunk[F * V])
    misc = misc.at[2, :HID].set(W_trunk[F * V + 1])
    misc = misc.at[3:3 + HID, :L].set(W_logits[:, 0:L])
    misc = misc.at[3:3 + HID, L:L + L].set(W_logits[:, 14:NUM_OUT])
    misc = misc.at[13, :L].set(b_logits[0:L])
    misc = misc.at[13, L:L + L].set(b_logits[14:NUM_OUT])
    misc = misc.at[14, :HID].set(W_value[:, 0])
    misc = misc.at[14, HID].set(b_value[0])
    logits_t, value = _pvnet_sc(x.T, ptcm, misc.reshape(16 * 32))
    return logits_t.T, value.reshape(B, 1)


# speedup vs baseline: 1.5470x; 1.0755x over previous
import functools

import jax
import jax.numpy as jnp
from jax import lax
from jax.experimental import pallas as pl
from jax.experimental.pallas import tpu as pltpu
from jax.experimental.pallas import tpu_sc as plsc

B = 16384
OBS = 80
F = 64
V = 8
HID = 10
NUM_OUT = 30

NC = 2
NS = 16
L = 16
NW = NC * NS
ROWS_PER_W = B // NW
CHUNKS = ROWS_PER_W // L
NPAIR = F // 2
PT_ROWS = NPAIR * V * V


def _body(xt_h, pt_h, misc_h, logits_h, value_h,
          xt_v, pt_v, misc_v, logits_v, value_v):
    wid = lax.axis_index("s") * NC + lax.axis_index("c")
    base = wid * ROWS_PER_W

    pltpu.sync_copy(xt_h.at[:, pl.ds(base, ROWS_PER_W)], xt_v)
    pltpu.sync_copy(pt_h, pt_v)
    pltpu.sync_copy(misc_h, misc_v)

    btr = misc_v[pl.ds(0 * 32, L)]
    w64 = misc_v[pl.ds(1 * 32, L)]
    w65 = misc_v[pl.ds(2 * 32, L)]
    whA = [misc_v[pl.ds((3 + c) * 32, L)] for c in range(HID)]
    whB = [misc_v[pl.ds((3 + c) * 32 + L, L)] for c in range(HID)]
    bhA = misc_v[pl.ds(13 * 32, L)]
    bhB = misc_v[pl.ds(13 * 32 + L, L)]
    wv = misc_v[pl.ds(14 * 32, L)]

    def chunk(i, carry):
        def xcol(c):
            return xt_v[c, pl.ds(i * L, L)]

        x64 = xcol(F)
        x65 = xcol(F + 1)
        acc = [btr[c] + x64 * w64[c] + x65 * w65[c] for c in range(HID)]
        for p in range(NPAIR):
            xe = xcol(2 * p).astype(jnp.int32)
            xo = xcol(2 * p + 1).astype(jnp.int32)
            ridx = ((p * V * V) * HID) + (xe * V + xo) * HID
            for c in range(HID):
                acc[c] = acc[c] + plsc.load_gather(pt_v, [ridx + c])
        def rbf(xv):
            u = plsc.bitcast(xv, jnp.uint32)
            r = (u + jnp.uint32(0x7FFF) + ((u >> jnp.uint32(16)) & jnp.uint32(1))) \
                & jnp.uint32(0xFFFF0000)
            return plsc.bitcast(r, jnp.float32)

        trunk = [rbf(jnp.maximum(a, 0.0)) for a in acc]
        for o in range(NUM_OUT):
            if o < L:
                lo = jnp.broadcast_to(bhA[o], (L,))
                for c in range(HID):
                    lo = lo + trunk[c] * jnp.broadcast_to(whA[c][o], (L,))
            else:
                lo = jnp.broadcast_to(bhB[o - 14], (L,))
                for c in range(HID):
                    lo = lo + trunk[c] * jnp.broadcast_to(whB[c][o - 14], (L,))
            logits_v[o, pl.ds(i * L, L)] = lo
        z = jnp.broadcast_to(wv[HID], (L,))
        for c in range(HID):
            z = z + trunk[c] * wv[c]
        e = jnp.exp(2.0 * z)
        value_v[pl.ds(i * L, L)] = (e - 1.0) / (e + 1.0)
        return carry

    lax.fori_loop(0, CHUNKS, chunk, 0)

    pltpu.sync_copy(logits_v, logits_h.at[:, pl.ds(base, ROWS_PER_W)])
    pltpu.sync_copy(value_v, value_h.at[pl.ds(base, ROWS_PER_W)])


@functools.partial(
    pl.kernel,
    out_type=(jax.ShapeDtypeStruct((NUM_OUT, B), jnp.float32),
              jax.ShapeDtypeStruct((B,), jnp.float32)),
    mesh=plsc.VectorSubcoreMesh(core_axis_name="c", subcore_axis_name="s",
                                num_cores=NC, num_subcores=NS),
    compiler_params=pltpu.CompilerParams(needs_layout_passes=False,
                                         use_tc_tiling_on_sc=True),
    scratch_types=[
        pltpu.VMEM((OBS, ROWS_PER_W), jnp.float32),
        pltpu.VMEM((PT_ROWS * HID,), jnp.float32),
        pltpu.VMEM((16 * 32,), jnp.float32),
        pltpu.VMEM((NUM_OUT, ROWS_PER_W), jnp.float32),
        pltpu.VMEM((ROWS_PER_W,), jnp.float32),
    ],
)
def _pvnet_sc(xt_h, pt_h, misc_h, logits_h, value_h,
              xt_v, pt_v, misc_v, logits_v, value_v):
    _body(xt_h, pt_h, misc_h, logits_h, value_h,
          xt_v, pt_v, misc_v, logits_v, value_v)


def kernel(x, one_hot_indices, identity_indices, values,
           W_trunk, b_trunk, W_logits, b_logits, W_value, b_value):
    W_trunk = W_trunk.astype(jnp.bfloat16).astype(jnp.float32)
    W_logits = W_logits.astype(jnp.bfloat16).astype(jnp.float32)
    W_value = W_value.astype(jnp.bfloat16).astype(jnp.float32)
    w3 = W_trunk[:F * V].reshape(F, V, HID)
    pt = (w3[0::2, :, None, :] + w3[1::2, None, :, :]).reshape(PT_ROWS * HID)
    misc = jnp.zeros((16, 32), jnp.float32)
    misc = misc.at[0, :HID].set(b_trunk)
    misc = misc.at[1, :HID].set(W_trunk[F * V])
    misc = misc.at[2, :HID].set(W_trunk[F * V + 1])
    misc = misc.at[3:3 + HID, :L].set(W_logits[:, 0:L])
    misc = misc.at[3:3 + HID, L:L + L].set(W_logits[:, 14:NUM_OUT])
    misc = misc.at[13, :L].set(b_logits[0:L])
    misc = misc.at[13, L:L + L].set(b_logits[14:NUM_OUT])
    misc = misc.at[14, :HID].set(W_value[:, 0])
    misc = misc.at[14, HID].set(b_value[0])
    logits_t, value = _pvnet_sc(x.T, pt, misc.reshape(16 * 32))
    return logits_t.T, value.reshape(B, 1)
